# baseline (speedup 1.0000x reference)
import functools

import jax
import jax.numpy as jnp
from jax import lax
from jax.experimental import pallas as pl
from jax.experimental.pallas import tpu as pltpu
from jax.experimental.pallas import tpu_sc as plsc

V = 1000000
D = 32
NNS1 = 5
B = 16384
CBo = 16384
QSTEP = 245760
RPQ = 278528
NBo = RPQ // CBo
NQB = QSTEP // CBo

NC = 2
NS = 16
NW = NC * NS
BPW = B // NW
QPW = BPW * 2
CPW = BPW * NNS1
WB = 64
NWAVE = BPW // WB
QW = WB * 2
CW = WB * NNS1


def _tp_body(in0, in1, in2, in3, out_ref):
    eye = jnp.eye(256, dtype=jnp.bfloat16)
    x = jnp.concatenate(
        [r[...].astype(jnp.bfloat16) for r in (in0, in1, in2, in3)], axis=0)
    for k in range(CBo // 256):
        xk = x[:, 256 * k:256 * (k + 1)]
        yk = jax.lax.dot_general(
            eye, xk, (((1,), (1,)), ((), ())),
            preferred_element_type=jnp.float32)
        out_ref[256 * k:256 * (k + 1), :] = yk


def _conv(tab_t):
    specs = [
        pl.BlockSpec((D, CBo), lambda j, a=a: (0, a * NQB + j))
        for a in range(4)
    ]
    return pl.pallas_call(
        _tp_body,
        grid=(NBo,),
        in_specs=specs,
        out_specs=pl.BlockSpec((CBo, 128), lambda j: (j, 0)),
        out_shape=jax.ShapeDtypeStruct((RPQ, 128), jnp.float32),
    )(tab_t, tab_t, tab_t, tab_t)


_mesh = plsc.VectorSubcoreMesh(
    core_axis_name="c", subcore_axis_name="s", num_cores=NC, num_subcores=NS)


@functools.partial(
    pl.kernel,
    out_type=jax.ShapeDtypeStruct((B * NNS1,), jnp.float32),
    mesh=_mesh,
    compiler_params=pltpu.CompilerParams(
        needs_layout_passes=False, use_tc_tiling_on_sc=False),
    scratch_types=[
        pltpu.VMEM((QPW,), jnp.int32),
        pltpu.VMEM((CPW,), jnp.int32),
        pltpu.VMEM((QW,), jnp.int32),
        pltpu.VMEM((QW,), jnp.int32),
        pltpu.VMEM((CW,), jnp.int32),
        pltpu.VMEM((CW,), jnp.int32),
        pltpu.VMEM((QW, 128), jnp.float32),
        pltpu.VMEM((QW, 128), jnp.float32),
        pltpu.VMEM((CW, 128), jnp.float32),
        pltpu.VMEM((CW, 128), jnp.float32),
        pltpu.VMEM((CPW,), jnp.float32),
        pltpu.SemaphoreType.DMA,
        pltpu.SemaphoreType.DMA,
    ],
)
def _w2v_sc(tgt_hbm, ctx_hbm, qpk_hbm, ppk_hbm, out_hbm,
            qidx_v, cidx_v, qm0_v, qm1_v, cm0_v, cm1_v,
            qr0_v, qr1_v, cr0_v, cr1_v, out_v, sem0, sem1):
    wid = lax.axis_index("s") * NC + lax.axis_index("c")

    pltpu.sync_copy(tgt_hbm.at[pl.ds(wid * QPW, QPW)], qidx_v)
    pltpu.sync_copy(ctx_hbm.at[pl.ds(wid * CPW, CPW)], cidx_v)

    iota16 = lax.iota(jnp.int32, 16)
    bufs = [(qm0_v, cm0_v, qr0_v, cr0_v, sem0),
            (qm1_v, cm1_v, qr1_v, cr1_v, sem1)]

    def fire(w, qm_v, cm_v, qr_v, cr_v, sem):
        three = jnp.full((16,), 3, jnp.int32)
        qstep = jnp.full((16,), QSTEP, jnp.int32)

        def qmod(i, c):
            v = qidx_v[pl.ds(w * QW + i * 16, 16)]
            a = jnp.minimum(lax.div(v, qstep), three)
            qm_v[pl.ds(i * 16, 16)] = v - a * QSTEP
            return c

        lax.fori_loop(0, QW // 16, qmod, 0)

        def cmod(i, c):
            v = cidx_v[pl.ds(w * CW + i * 16, 16)]
            a = jnp.minimum(lax.div(v, qstep), three)
            cm_v[pl.ds(i * 16, 16)] = v - a * QSTEP
            return c

        lax.fori_loop(0, CW // 16, cmod, 0)

        copies = [pltpu.async_copy(qpk_hbm.at[qm_v], qr_v, sem)]
        for i in range((CW + 127) // 128):
            n = min(128, CW - i * 128)
            copies.append(pltpu.async_copy(
                ppk_hbm.at[cm_v.at[pl.ds(i * 128, n)]],
                cr_v.at[pl.ds(i * 128, n)], sem))
        return copies

    def compute(w, qr_v, cr_v):
        def gstep(g, carry2):
            lb = g * 16 + iota16
            q0 = 2 * lb
            q1 = q0 + 1
            qs16 = jnp.full((16,), QSTEP, jnp.int32)
            th16 = jnp.full((16,), 3, jnp.int32)
            r0 = plsc.load_gather(qidx_v, [w * QW + q0])
            r1 = plsc.load_gather(qidx_v, [w * QW + q1])
            qc0 = 32 * jnp.minimum(lax.div(r0, qs16), th16)
            qc1 = 32 * jnp.minimum(lax.div(r1, qs16), th16)
            cix = [NNS1 * lb + n for n in range(NNS1)]
            ccs = []
            for n in range(NNS1):
                rc = plsc.load_gather(cidx_v, [w * CW + cix[n]])
                ccs.append(32 * jnp.minimum(lax.div(rc, qs16), th16))
            acc = [jnp.zeros((16,), jnp.float32) for _ in range(NNS1)]
            for d in range(D):
                wv = (plsc.load_gather(qr_v, [q0, qc0 + d])
                      + plsc.load_gather(qr_v, [q1, qc1 + d]))
                for n in range(NNS1):
                    acc[n] = acc[n] + plsc.load_gather(
                        cr_v, [cix[n], ccs[n] + d]) * wv
            obase = w * CW
            for n in range(NNS1):
                plsc.store_scatter(out_v, [obase + cix[n]], acc[n] * 0.5)
            return carry2

        lax.fori_loop(0, WB // 16, gstep, 0)

    def wait_wave(qm_v, cm_v, qr_v, cr_v, sem):
        pltpu.make_async_copy(qpk_hbm.at[qm_v], qr_v, sem).wait()
        for i in range((CW + 127) // 128):
            n = min(128, CW - i * 128)
            pltpu.make_async_copy(
                ppk_hbm.at[cm_v.at[pl.ds(i * 128, n)]],
                cr_v.at[pl.ds(i * 128, n)], sem).wait()

    fire(0, *bufs[0])
    fire(1, *bufs[1])

    def w2body(w2, carry):
        for b in range(2):
            w = 2 * w2 + b
            wait_wave(*bufs[b])
            compute(w, bufs[b][2], bufs[b][3])
            fire(lax.rem(w + 2, NWAVE), *bufs[b])
        return carry

    lax.fori_loop(0, NWAVE // 2, w2body, 0)
    for b in range(2):
        wait_wave(*bufs[b])

    pltpu.sync_copy(out_v, out_hbm.at[pl.ds(wid * CPW, CPW)])


def kernel(target, context, query_table, poi_table):
    qpk = _conv(query_table.T)
    ppk = _conv(poi_table.T)
    out = _w2v_sc(target.reshape(-1), context.reshape(-1), qpk, ppk)
    return out.reshape(B, NNS1)


# speedup vs baseline: 1.0061x; 1.0061x over previous
import functools

import jax
import jax.numpy as jnp
from jax import lax
from jax.experimental import pallas as pl
from jax.experimental.pallas import tpu as pltpu
from jax.experimental.pallas import tpu_sc as plsc

V = 1000000
D = 32
NNS1 = 5
B = 16384
CBo = 16384
QSTEP = 245760
RPQ = 278528
NBo = RPQ // CBo
NQB = QSTEP // CBo

NC = 2
NS = 16
NW = NC * NS
BPW = B // NW
QPW = BPW * 2
CPW = BPW * NNS1
WB = 32
NWAVE = BPW // WB
QW = WB * 2
CW = WB * NNS1


def _tp_body(in0, in1, in2, in3, out_ref):
    eye = jnp.eye(256, dtype=jnp.bfloat16)
    x = jnp.concatenate(
        [r[...].astype(jnp.bfloat16) for r in (in0, in1, in2, in3)], axis=0)
    for k in range(CBo // 256):
        xk = x[:, 256 * k:256 * (k + 1)]
        yk = jax.lax.dot_general(
            eye, xk, (((1,), (1,)), ((), ())),
            preferred_element_type=jnp.float32)
        out_ref[256 * k:256 * (k + 1), :] = yk


def _conv(tab_t):
    specs = [
        pl.BlockSpec((D, CBo), lambda j, a=a: (0, a * NQB + j))
        for a in range(4)
    ]
    return pl.pallas_call(
        _tp_body,
        grid=(NBo,),
        in_specs=specs,
        out_specs=pl.BlockSpec((CBo, 128), lambda j: (j, 0)),
        out_shape=jax.ShapeDtypeStruct((RPQ, 128), jnp.float32),
    )(tab_t, tab_t, tab_t, tab_t)


_mesh = plsc.VectorSubcoreMesh(
    core_axis_name="c", subcore_axis_name="s", num_cores=NC, num_subcores=NS)


@functools.partial(
    pl.kernel,
    out_type=jax.ShapeDtypeStruct((B * NNS1,), jnp.float32),
    mesh=_mesh,
    compiler_params=pltpu.CompilerParams(
        needs_layout_passes=False, use_tc_tiling_on_sc=False),
    scratch_types=[
        pltpu.VMEM((QPW,), jnp.int32),
        pltpu.VMEM((CPW,), jnp.int32),
        pltpu.VMEM((QW,), jnp.int32),
        pltpu.VMEM((QW,), jnp.int32),
        pltpu.VMEM((CW,), jnp.int32),
        pltpu.VMEM((CW,), jnp.int32),
        pltpu.VMEM((QW, 128), jnp.float32),
        pltpu.VMEM((QW, 128), jnp.float32),
        pltpu.VMEM((CW, 128), jnp.float32),
        pltpu.VMEM((CW, 128), jnp.float32),
        pltpu.VMEM((CPW,), jnp.float32),
        pltpu.SemaphoreType.DMA,
        pltpu.SemaphoreType.DMA,
    ],
)
def _w2v_sc(tgt_hbm, ctx_hbm, qpk_hbm, ppk_hbm, out_hbm,
            qidx_v, cidx_v, qm0_v, qm1_v, cm0_v, cm1_v,
            qr0_v, qr1_v, cr0_v, cr1_v, out_v, sem0, sem1):
    wid = lax.axis_index("s") * NC + lax.axis_index("c")

    pltpu.sync_copy(tgt_hbm.at[pl.ds(wid * QPW, QPW)], qidx_v)
    pltpu.sync_copy(ctx_hbm.at[pl.ds(wid * CPW, CPW)], cidx_v)

    iota16 = lax.iota(jnp.int32, 16)
    bufs = [(qm0_v, cm0_v, qr0_v, cr0_v, sem0),
            (qm1_v, cm1_v, qr1_v, cr1_v, sem1)]

    def fire(w, qm_v, cm_v, qr_v, cr_v, sem):
        three = jnp.full((16,), 3, jnp.int32)
        qstep = jnp.full((16,), QSTEP, jnp.int32)

        def qmod(i, c):
            v = qidx_v[pl.ds(w * QW + i * 16, 16)]
            a = jnp.minimum(lax.div(v, qstep), three)
            qm_v[pl.ds(i * 16, 16)] = v - a * QSTEP
            return c

        lax.fori_loop(0, QW // 16, qmod, 0)

        def cmod(i, c):
            v = cidx_v[pl.ds(w * CW + i * 16, 16)]
            a = jnp.minimum(lax.div(v, qstep), three)
            cm_v[pl.ds(i * 16, 16)] = v - a * QSTEP
            return c

        lax.fori_loop(0, CW // 16, cmod, 0)

        copies = [pltpu.async_copy(qpk_hbm.at[qm_v], qr_v, sem)]
        for i in range((CW + 127) // 128):
            n = min(128, CW - i * 128)
            copies.append(pltpu.async_copy(
                ppk_hbm.at[cm_v.at[pl.ds(i * 128, n)]],
                cr_v.at[pl.ds(i * 128, n)], sem))
        return copies

    def compute(w, qr_v, cr_v):
        def gstep(g, carry2):
            lb = g * 16 + iota16
            q0 = 2 * lb
            q1 = q0 + 1
            qs16 = jnp.full((16,), QSTEP, jnp.int32)
            th16 = jnp.full((16,), 3, jnp.int32)
            r0 = plsc.load_gather(qidx_v, [w * QW + q0])
            r1 = plsc.load_gather(qidx_v, [w * QW + q1])
            qc0 = 32 * jnp.minimum(lax.div(r0, qs16), th16)
            qc1 = 32 * jnp.minimum(lax.div(r1, qs16), th16)
            cix = [NNS1 * lb + n for n in range(NNS1)]
            ccs = []
            for n in range(NNS1):
                rc = plsc.load_gather(cidx_v, [w * CW + cix[n]])
                ccs.append(32 * jnp.minimum(lax.div(rc, qs16), th16))
            acc = [jnp.zeros((16,), jnp.float32) for _ in range(NNS1)]
            for d in range(D):
                wv = (plsc.load_gather(qr_v, [q0, qc0 + d])
                      + plsc.load_gather(qr_v, [q1, qc1 + d]))
                for n in range(NNS1):
                    acc[n] = acc[n] + plsc.load_gather(
                        cr_v, [cix[n], ccs[n] + d]) * wv
            obase = w * CW
            for n in range(NNS1):
                plsc.store_scatter(out_v, [obase + cix[n]], acc[n] * 0.5)
            return carry2

        lax.fori_loop(0, WB // 16, gstep, 0)

    def wait_wave(qm_v, cm_v, qr_v, cr_v, sem):
        pltpu.make_async_copy(qpk_hbm.at[qm_v], qr_v, sem).wait()
        for i in range((CW + 127) // 128):
            n = min(128, CW - i * 128)
            pltpu.make_async_copy(
                ppk_hbm.at[cm_v.at[pl.ds(i * 128, n)]],
                cr_v.at[pl.ds(i * 128, n)], sem).wait()

    fire(0, *bufs[0])
    fire(1, *bufs[1])

    def w2body(w2, carry):
        for b in range(2):
            w = 2 * w2 + b
            wait_wave(*bufs[b])
            compute(w, bufs[b][2], bufs[b][3])
            fire(lax.rem(w + 2, NWAVE), *bufs[b])
        return carry

    lax.fori_loop(0, NWAVE // 2, w2body, 0)
    for b in range(2):
        wait_wave(*bufs[b])

    pltpu.sync_copy(out_v, out_hbm.at[pl.ds(wid * CPW, CPW)])


def kernel(target, context, query_table, poi_table):
    qpk = _conv(query_table.T)
    ppk = _conv(poi_table.T)
    out = _w2v_sc(target.reshape(-1), context.reshape(-1), qpk, ppk)
    return out.reshape(B, NNS1)


# speedup vs baseline: 1.0368x; 1.0305x over previous
import functools

import jax
import jax.numpy as jnp
from jax import lax
from jax.experimental import pallas as pl
from jax.experimental.pallas import tpu as pltpu
from jax.experimental.pallas import tpu_sc as plsc

V = 1000000
D = 32
NNS1 = 5
B = 16384
CBo = 8192
QSTEP = 245760
RPQ = 270336
NBo = RPQ // CBo
NQB = QSTEP // CBo

NC = 2
NS = 16
NW = NC * NS
BPW = B // NW
QPW = BPW * 2
CPW = BPW * NNS1
WB = 32
NWAVE = BPW // WB
QW = WB * 2
CW = WB * NNS1


def _tp_half(ins, out_ref):
    eye = jnp.eye(256, dtype=jnp.bfloat16)
    x = jnp.concatenate([r[...].astype(jnp.bfloat16) for r in ins], axis=0)
    for k in range(CBo // 256):
        xk = x[:, 256 * k:256 * (k + 1)]
        yk = jax.lax.dot_general(
            eye, xk, (((1,), (1,)), ((), ())),
            preferred_element_type=jnp.float32)
        out_ref[256 * k:256 * (k + 1), :] = yk


def _tp_body(q0, q1, q2, q3, p0, p1, p2, p3, qout_ref, pout_ref):
    _tp_half((q0, q1, q2, q3), qout_ref)
    _tp_half((p0, p1, p2, p3), pout_ref)


def _conv2(qtab_t, ptab_t):
    specs = [
        pl.BlockSpec((D, CBo), lambda j, a=a: (0, a * NQB + j))
        for a in range(4)
    ]
    out_spec = pl.BlockSpec((CBo, 128), lambda j: (j, 0))
    return pl.pallas_call(
        _tp_body,
        grid=(NBo,),
        in_specs=specs + specs,
        out_specs=[out_spec, out_spec],
        out_shape=[jax.ShapeDtypeStruct((RPQ, 128), jnp.float32)] * 2,
    )(qtab_t, qtab_t, qtab_t, qtab_t, ptab_t, ptab_t, ptab_t, ptab_t)


_mesh = plsc.VectorSubcoreMesh(
    core_axis_name="c", subcore_axis_name="s", num_cores=NC, num_subcores=NS)


@functools.partial(
    pl.kernel,
    out_type=jax.ShapeDtypeStruct((B * NNS1,), jnp.float32),
    mesh=_mesh,
    compiler_params=pltpu.CompilerParams(
        needs_layout_passes=False, use_tc_tiling_on_sc=False),
    scratch_types=[
        pltpu.VMEM((QPW,), jnp.int32),
        pltpu.VMEM((CPW,), jnp.int32),
        pltpu.VMEM((QW,), jnp.int32),
        pltpu.VMEM((QW,), jnp.int32),
        pltpu.VMEM((CW,), jnp.int32),
        pltpu.VMEM((CW,), jnp.int32),
        pltpu.VMEM((QW, 128), jnp.float32),
        pltpu.VMEM((QW, 128), jnp.float32),
        pltpu.VMEM((CW, 128), jnp.float32),
        pltpu.VMEM((CW, 128), jnp.float32),
        pltpu.VMEM((CPW,), jnp.float32),
        pltpu.SemaphoreType.DMA,
        pltpu.SemaphoreType.DMA,
    ],
)
def _w2v_sc(tgt_hbm, ctx_hbm, qpk_hbm, ppk_hbm, out_hbm,
            qidx_v, cidx_v, qm0_v, qm1_v, cm0_v, cm1_v,
            qr0_v, qr1_v, cr0_v, cr1_v, out_v, sem0, sem1):
    wid = lax.axis_index("s") * NC + lax.axis_index("c")

    pltpu.sync_copy(tgt_hbm.at[pl.ds(wid * QPW, QPW)], qidx_v)
    pltpu.sync_copy(ctx_hbm.at[pl.ds(wid * CPW, CPW)], cidx_v)

    iota16 = lax.iota(jnp.int32, 16)
    bufs = [(qm0_v, cm0_v, qr0_v, cr0_v, sem0),
            (qm1_v, cm1_v, qr1_v, cr1_v, sem1)]

    def fire(w, qm_v, cm_v, qr_v, cr_v, sem):
        three = jnp.full((16,), 3, jnp.int32)
        qstep = jnp.full((16,), QSTEP, jnp.int32)

        def qmod(i, c):
            v = qidx_v[pl.ds(w * QW + i * 16, 16)]
            a = jnp.minimum(lax.div(v, qstep), three)
            qm_v[pl.ds(i * 16, 16)] = v - a * QSTEP
            return c

        lax.fori_loop(0, QW // 16, qmod, 0)

        def cmod(i, c):
            v = cidx_v[pl.ds(w * CW + i * 16, 16)]
            a = jnp.minimum(lax.div(v, qstep), three)
            cm_v[pl.ds(i * 16, 16)] = v - a * QSTEP
            return c

        lax.fori_loop(0, CW // 16, cmod, 0)

        copies = [pltpu.async_copy(qpk_hbm.at[qm_v], qr_v, sem)]
        for i in range((CW + 127) // 128):
            n = min(128, CW - i * 128)
            copies.append(pltpu.async_copy(
                ppk_hbm.at[cm_v.at[pl.ds(i * 128, n)]],
                cr_v.at[pl.ds(i * 128, n)], sem))
        return copies

    def compute(w, qr_v, cr_v):
        def gstep(g, carry2):
            lb = g * 16 + iota16
            q0 = 2 * lb
            q1 = q0 + 1
            qs16 = jnp.full((16,), QSTEP, jnp.int32)
            th16 = jnp.full((16,), 3, jnp.int32)
            r0 = plsc.load_gather(qidx_v, [w * QW + q0])
            r1 = plsc.load_gather(qidx_v, [w * QW + q1])
            qc0 = 32 * jnp.minimum(lax.div(r0, qs16), th16)
            qc1 = 32 * jnp.minimum(lax.div(r1, qs16), th16)
            cix = [NNS1 * lb + n for n in range(NNS1)]
            ccs = []
            for n in range(NNS1):
                rc = plsc.load_gather(cidx_v, [w * CW + cix[n]])
                ccs.append(32 * jnp.minimum(lax.div(rc, qs16), th16))
            acc = [jnp.zeros((16,), jnp.float32) for _ in range(NNS1)]
            for d in range(D):
                wv = (plsc.load_gather(qr_v, [q0, qc0 + d])
                      + plsc.load_gather(qr_v, [q1, qc1 + d]))
                for n in range(NNS1):
                    acc[n] = acc[n] + plsc.load_gather(
                        cr_v, [cix[n], ccs[n] + d]) * wv
            obase = w * CW
            for n in range(NNS1):
                plsc.store_scatter(out_v, [obase + cix[n]], acc[n] * 0.5)
            return carry2

        lax.fori_loop(0, WB // 16, gstep, 0)

    def wait_wave(qm_v, cm_v, qr_v, cr_v, sem):
        pltpu.make_async_copy(qpk_hbm.at[qm_v], qr_v, sem).wait()
        for i in range((CW + 127) // 128):
            n = min(128, CW - i * 128)
            pltpu.make_async_copy(
                ppk_hbm.at[cm_v.at[pl.ds(i * 128, n)]],
                cr_v.at[pl.ds(i * 128, n)], sem).wait()

    fire(0, *bufs[0])
    fire(1, *bufs[1])

    def w2body(w2, carry):
        for b in range(2):
            w = 2 * w2 + b
            wait_wave(*bufs[b])
            compute(w, bufs[b][2], bufs[b][3])
            fire(lax.rem(w + 2, NWAVE), *bufs[b])
        return carry

    lax.fori_loop(0, NWAVE // 2, w2body, 0)
    for b in range(2):
        wait_wave(*bufs[b])

    pltpu.sync_copy(out_v, out_hbm.at[pl.ds(wid * CPW, CPW)])


def kernel(target, context, query_table, poi_table):
    qpk, ppk = _conv2(query_table.T, poi_table.T)
    out = _w2v_sc(target.reshape(-1), context.reshape(-1), qpk, ppk)
    return out.reshape(B, NNS1)


# speedup vs baseline: 1.0419x; 1.0050x over previous
import functools

import jax
import jax.numpy as jnp
from jax import lax
from jax.experimental import pallas as pl
from jax.experimental.pallas import tpu as pltpu
from jax.experimental.pallas import tpu_sc as plsc

V = 1000000
D = 32
NNS1 = 5
B = 16384
CBo = 12288
QSTEP = 245760
RPQ = 270336
NBo = RPQ // CBo
NQB = QSTEP // CBo

NC = 2
NS = 16
NW = NC * NS
BPW = B // NW
QPW = BPW * 2
CPW = BPW * NNS1
WB = 32
NWAVE = BPW // WB
QW = WB * 2
CW = WB * NNS1


def _tp_half(ins, out_ref):
    eye = jnp.eye(256, dtype=jnp.bfloat16)
    x = jnp.concatenate([r[...].astype(jnp.bfloat16) for r in ins], axis=0)
    for k in range(CBo // 256):
        xk = x[:, 256 * k:256 * (k + 1)]
        yk = jax.lax.dot_general(
            eye, xk, (((1,), (1,)), ((), ())),
            preferred_element_type=jnp.float32)
        out_ref[256 * k:256 * (k + 1), :] = yk


def _tp_body(q0, q1, q2, q3, p0, p1, p2, p3, qout_ref, pout_ref):
    _tp_half((q0, q1, q2, q3), qout_ref)
    _tp_half((p0, p1, p2, p3), pout_ref)


def _conv2(qtab_t, ptab_t):
    specs = [
        pl.BlockSpec((D, CBo), lambda j, a=a: (0, a * NQB + j))
        for a in range(4)
    ]
    out_spec = pl.BlockSpec((CBo, 128), lambda j: (j, 0))
    return pl.pallas_call(
        _tp_body,
        grid=(NBo,),
        in_specs=specs + specs,
        out_specs=[out_spec, out_spec],
        out_shape=[jax.ShapeDtypeStruct((RPQ, 128), jnp.float32)] * 2,
    )(qtab_t, qtab_t, qtab_t, qtab_t, ptab_t, ptab_t, ptab_t, ptab_t)


_mesh = plsc.VectorSubcoreMesh(
    core_axis_name="c", subcore_axis_name="s", num_cores=NC, num_subcores=NS)


@functools.partial(
    pl.kernel,
    out_type=jax.ShapeDtypeStruct((B * NNS1,), jnp.float32),
    mesh=_mesh,
    compiler_params=pltpu.CompilerParams(
        needs_layout_passes=False, use_tc_tiling_on_sc=False),
    scratch_types=[
        pltpu.VMEM((QPW,), jnp.int32),
        pltpu.VMEM((CPW,), jnp.int32),
        pltpu.VMEM((QW,), jnp.int32),
        pltpu.VMEM((QW,), jnp.int32),
        pltpu.VMEM((CW,), jnp.int32),
        pltpu.VMEM((CW,), jnp.int32),
        pltpu.VMEM((QW, 128), jnp.float32),
        pltpu.VMEM((QW, 128), jnp.float32),
        pltpu.VMEM((CW, 128), jnp.float32),
        pltpu.VMEM((CW, 128), jnp.float32),
        pltpu.VMEM((CPW,), jnp.float32),
        pltpu.SemaphoreType.DMA,
        pltpu.SemaphoreType.DMA,
    ],
)
def _w2v_sc(tgt_hbm, ctx_hbm, qpk_hbm, ppk_hbm, out_hbm,
            qidx_v, cidx_v, qm0_v, qm1_v, cm0_v, cm1_v,
            qr0_v, qr1_v, cr0_v, cr1_v, out_v, sem0, sem1):
    wid = lax.axis_index("s") * NC + lax.axis_index("c")

    pltpu.sync_copy(tgt_hbm.at[pl.ds(wid * QPW, QPW)], qidx_v)
    pltpu.sync_copy(ctx_hbm.at[pl.ds(wid * CPW, CPW)], cidx_v)

    iota16 = lax.iota(jnp.int32, 16)
    bufs = [(qm0_v, cm0_v, qr0_v, cr0_v, sem0),
            (qm1_v, cm1_v, qr1_v, cr1_v, sem1)]

    def fire(w, qm_v, cm_v, qr_v, cr_v, sem):
        three = jnp.full((16,), 3, jnp.int32)
        qstep = jnp.full((16,), QSTEP, jnp.int32)

        def qmod(i, c):
            v = qidx_v[pl.ds(w * QW + i * 16, 16)]
            a = jnp.minimum(lax.div(v, qstep), three)
            qm_v[pl.ds(i * 16, 16)] = v - a * QSTEP
            return c

        lax.fori_loop(0, QW // 16, qmod, 0)

        def cmod(i, c):
            v = cidx_v[pl.ds(w * CW + i * 16, 16)]
            a = jnp.minimum(lax.div(v, qstep), three)
            cm_v[pl.ds(i * 16, 16)] = v - a * QSTEP
            return c

        lax.fori_loop(0, CW // 16, cmod, 0)

        copies = [pltpu.async_copy(qpk_hbm.at[qm_v], qr_v, sem)]
        for i in range((CW + 127) // 128):
            n = min(128, CW - i * 128)
            copies.append(pltpu.async_copy(
                ppk_hbm.at[cm_v.at[pl.ds(i * 128, n)]],
                cr_v.at[pl.ds(i * 128, n)], sem))
        return copies

    def compute(w, qr_v, cr_v):
        def gstep(g, carry2):
            lb = g * 16 + iota16
            q0 = 2 * lb
            q1 = q0 + 1
            qs16 = jnp.full((16,), QSTEP, jnp.int32)
            th16 = jnp.full((16,), 3, jnp.int32)
            r0 = plsc.load_gather(qidx_v, [w * QW + q0])
            r1 = plsc.load_gather(qidx_v, [w * QW + q1])
            qc0 = 32 * jnp.minimum(lax.div(r0, qs16), th16)
            qc1 = 32 * jnp.minimum(lax.div(r1, qs16), th16)
            cix = [NNS1 * lb + n for n in range(NNS1)]
            ccs = []
            for n in range(NNS1):
                rc = plsc.load_gather(cidx_v, [w * CW + cix[n]])
                ccs.append(32 * jnp.minimum(lax.div(rc, qs16), th16))
            acc = [jnp.zeros((16,), jnp.float32) for _ in range(NNS1)]
            for d in range(D):
                wv = (plsc.load_gather(qr_v, [q0, qc0 + d])
                      + plsc.load_gather(qr_v, [q1, qc1 + d]))
                for n in range(NNS1):
                    acc[n] = acc[n] + plsc.load_gather(
                        cr_v, [cix[n], ccs[n] + d]) * wv
            obase = w * CW
            for n in range(NNS1):
                plsc.store_scatter(out_v, [obase + cix[n]], acc[n] * 0.5)
            return carry2

        lax.fori_loop(0, WB // 16, gstep, 0)

    def wait_wave(qm_v, cm_v, qr_v, cr_v, sem):
        pltpu.make_async_copy(qpk_hbm.at[qm_v], qr_v, sem).wait()
        for i in range((CW + 127) // 128):
            n = min(128, CW - i * 128)
            pltpu.make_async_copy(
                ppk_hbm.at[cm_v.at[pl.ds(i * 128, n)]],
                cr_v.at[pl.ds(i * 128, n)], sem).wait()

    fire(0, *bufs[0])
    fire(1, *bufs[1])

    def w2body(w2, carry):
        for b in range(2):
            w = 2 * w2 + b
            wait_wave(*bufs[b])
            compute(w, bufs[b][2], bufs[b][3])
            fire(lax.rem(w + 2, NWAVE), *bufs[b])
        return carry

    lax.fori_loop(0, NWAVE // 2, w2body, 0)
    for b in range(2):
        wait_wave(*bufs[b])

    pltpu.sync_copy(out_v, out_hbm.at[pl.ds(wid * CPW, CPW)])


def kernel(target, context, query_table, poi_table):
    qpk, ppk = _conv2(query_table.T, poi_table.T)
    out = _w2v_sc(target.reshape(-1), context.reshape(-1), qpk, ppk)
    return out.reshape(B, NNS1)
